# baseline (speedup 1.0000x reference)
import jax
import jax.numpy as jnp
from jax.experimental import pallas as pl
from jax.experimental.pallas import tpu as pltpu

_ALPHA = 0.1
_BETA = 0.5
_BM = 400
_BM2 = 1000
_F8 = jnp.float8_e4m3fn
_S2_SCALE = 1.0 / 64.0
_PAD = 8


def _layer_kernel(x_ref, w1_ref, b1_ref, adj_ref, wc_ref,
                  o_ref, a8_ref, s8_ref, d_ref, h_ref):
    i = pl.program_id(0)

    @pl.when(i == 0)
    def _():
        hv = jnp.dot(x_ref[...].astype(jnp.bfloat16),
                     w1_ref[...].astype(jnp.bfloat16),
                     preferred_element_type=jnp.float32)
        h_ref[...] = jnp.maximum(hv + b1_ref[...], 0.0).astype(jnp.bfloat16)

    a = adj_ref[...]
    a8_ref[...] = a.astype(_F8)
    h_blk = h_ref[pl.ds(i * a.shape[0], a.shape[0]), :]
    agg = jnp.dot(a.astype(jnp.bfloat16), h_ref[...],
                  preferred_element_type=jnp.float32)
    mix = jnp.dot(h_blk, wc_ref[...].astype(jnp.bfloat16),
                  preferred_element_type=jnp.float32)
    out = jnp.maximum((1.0 - _BETA) * agg + _BETA * mix, 0.0)
    s2 = ((1.0 - _ALPHA) * out
          + _ALPHA * h_blk.astype(jnp.float32))
    o_ref[...] = s2.astype(jnp.bfloat16)

    s8 = (s2 * _S2_SCALE).astype(_F8)
    bm = s2.shape[0]
    probe = jnp.where(
        jax.lax.broadcasted_iota(jnp.int32, (bm, _PAD), 1) == 0,
        jnp.float32(_S2_SCALE), 0.0).astype(_F8)
    s8_ref[...] = jnp.concatenate([s8, probe], axis=1)

    delta = s2 - s8.astype(jnp.float32) * (1.0 / _S2_SCALE)
    dcol = jnp.sum(delta, axis=0, keepdims=True)

    @pl.when(i == 0)
    def _():
        d_ref[...] = dcol

    @pl.when(i > 0)
    def _():
        d_ref[...] += dcol


def _final_kernel(a8_ref, sup8_ref, s_blk_ref, d_ref, wc_ref, w2_ref, b2_ref,
                  o_ref):
    f = s_blk_ref.shape[1]
    agg_ext = jnp.dot(a8_ref[...], sup8_ref[...],
                      preferred_element_type=jnp.float32) * (1.0 / _S2_SCALE)
    rowsum = agg_ext[:, f:f + 1]
    n = a8_ref.shape[1]
    agg = agg_ext[:, :f] + rowsum * (d_ref[...] * (1.0 / n))
    mix = jnp.dot(s_blk_ref[...], wc_ref[...].astype(jnp.bfloat16),
                  preferred_element_type=jnp.float32)
    h2 = jnp.maximum((1.0 - _BETA) * agg + _BETA * mix, 0.0)
    logits = jnp.dot(h2.astype(jnp.bfloat16), w2_ref[...].astype(jnp.bfloat16),
                     preferred_element_type=jnp.float32) + b2_ref[...]
    m = jnp.max(logits, axis=1, keepdims=True)
    lse = m + jnp.log(jnp.sum(jnp.exp(logits - m), axis=1, keepdims=True))
    o_ref[...] = logits - lse


def kernel(x, adj, W1, b1, Wc1, Wc2, W2, b2):
    N, F = x.shape
    C = W2.shape[1]
    grid = (N // _BM,)

    row_blk = pl.BlockSpec((_BM, F), lambda i: (i, 0))
    adj_blk = pl.BlockSpec((_BM, N), lambda i: (i, 0))
    full = lambda shape: pl.BlockSpec(shape, lambda i: (0, 0))

    s2, adj8, s2_8, d = pl.pallas_call(
        _layer_kernel,
        grid=grid,
        in_specs=[full((N, F)), full((F, F)), full((1, F)),
                  adj_blk, full((F, F))],
        out_specs=(row_blk, adj_blk,
                   pl.BlockSpec((_BM, F + _PAD), lambda i: (i, 0)),
                   full((1, F))),
        out_shape=(jax.ShapeDtypeStruct((N, F), jnp.bfloat16),
                   jax.ShapeDtypeStruct((N, N), _F8),
                   jax.ShapeDtypeStruct((N, F + _PAD), _F8),
                   jax.ShapeDtypeStruct((1, F), jnp.float32)),
        scratch_shapes=[pltpu.VMEM((N, F), jnp.bfloat16)],
    )(x, W1, b1.reshape(1, F), adj, Wc1)

    out = pl.pallas_call(
        _final_kernel,
        grid=(N // _BM2,),
        in_specs=[pl.BlockSpec((_BM2, N), lambda i: (i, 0)),
                  full((N, F + _PAD)),
                  pl.BlockSpec((_BM2, F), lambda i: (i, 0)),
                  full((1, F)),
                  full((F, F)), full((F, C)), full((1, C))],
        out_specs=pl.BlockSpec((_BM2, C), lambda i: (i, 0)),
        out_shape=jax.ShapeDtypeStruct((N, C), jnp.float32),
    )(adj8, s2_8, s2, d, Wc2, W2, b2.reshape(1, C))

    return out


# speedup vs baseline: 1.0204x; 1.0204x over previous
import jax
import jax.numpy as jnp
from jax.experimental import pallas as pl
from jax.experimental.pallas import tpu as pltpu

_ALPHA = 0.1
_BETA = 0.5
_BM = 400
_BM2 = 1000
_F8 = jnp.float8_e4m3fn
_S2_SCALE = 1.0 / 64.0
_PAD = 8


def _layer_kernel(x_ref, w1_ref, b1_ref, adj_ref, wc_ref,
                  a8_ref, s8_ref, d_ref, h_ref):
    i = pl.program_id(0)

    @pl.when(i == 0)
    def _():
        hv = jnp.dot(x_ref[...].astype(jnp.bfloat16),
                     w1_ref[...].astype(jnp.bfloat16),
                     preferred_element_type=jnp.float32)
        h_ref[...] = jnp.maximum(hv + b1_ref[...], 0.0).astype(jnp.bfloat16)

    a = adj_ref[...]
    a8_ref[...] = a.astype(_F8)
    h_blk = h_ref[pl.ds(i * a.shape[0], a.shape[0]), :]
    agg = jnp.dot(a.astype(jnp.bfloat16), h_ref[...],
                  preferred_element_type=jnp.float32)
    mix = jnp.dot(h_blk, wc_ref[...].astype(jnp.bfloat16),
                  preferred_element_type=jnp.float32)
    out = jnp.maximum((1.0 - _BETA) * agg + _BETA * mix, 0.0)
    s2 = ((1.0 - _ALPHA) * out
          + _ALPHA * h_blk.astype(jnp.float32))

    s8 = (s2 * _S2_SCALE).astype(_F8)
    bm = s2.shape[0]
    probe = jnp.where(
        jax.lax.broadcasted_iota(jnp.int32, (bm, _PAD), 1) == 0,
        jnp.float32(_S2_SCALE), 0.0).astype(_F8)
    s8_ref[...] = jnp.concatenate([s8, probe], axis=1)

    delta = s2 - s8.astype(jnp.float32) * (1.0 / _S2_SCALE)
    dcol = jnp.sum(delta, axis=0, keepdims=True)

    @pl.when(i == 0)
    def _():
        d_ref[...] = dcol

    @pl.when(i > 0)
    def _():
        d_ref[...] += dcol


def _final_kernel(a8_ref, sup8_ref, d_ref, wc_ref, w2_ref, b2_ref,
                  o_ref):
    f = wc_ref.shape[0]
    bm = o_ref.shape[0]
    i = pl.program_id(0)
    agg_ext = jnp.dot(a8_ref[...], sup8_ref[...],
                      preferred_element_type=jnp.float32) * (1.0 / _S2_SCALE)
    rowsum = agg_ext[:, f:f + 1]
    n = a8_ref.shape[1]
    agg = agg_ext[:, :f] + rowsum * (d_ref[...] * (1.0 / n))
    s_blk = sup8_ref[pl.ds(i * bm, bm), :f].astype(jnp.bfloat16)
    mix = jnp.dot(s_blk, wc_ref[...].astype(jnp.bfloat16),
                  preferred_element_type=jnp.float32) * (1.0 / _S2_SCALE)
    h2 = jnp.maximum((1.0 - _BETA) * agg + _BETA * mix, 0.0)
    logits = jnp.dot(h2.astype(jnp.bfloat16), w2_ref[...].astype(jnp.bfloat16),
                     preferred_element_type=jnp.float32) + b2_ref[...]
    m = jnp.max(logits, axis=1, keepdims=True)
    lse = m + jnp.log(jnp.sum(jnp.exp(logits - m), axis=1, keepdims=True))
    o_ref[...] = logits - lse


def kernel(x, adj, W1, b1, Wc1, Wc2, W2, b2):
    N, F = x.shape
    C = W2.shape[1]
    grid = (N // _BM,)

    row_blk = pl.BlockSpec((_BM, F), lambda i: (i, 0))
    adj_blk = pl.BlockSpec((_BM, N), lambda i: (i, 0))
    full = lambda shape: pl.BlockSpec(shape, lambda i: (0, 0))

    adj8, s2_8, d = pl.pallas_call(
        _layer_kernel,
        grid=grid,
        in_specs=[full((N, F)), full((F, F)), full((1, F)),
                  adj_blk, full((F, F))],
        out_specs=(adj_blk,
                   pl.BlockSpec((_BM, F + _PAD), lambda i: (i, 0)),
                   full((1, F))),
        out_shape=(jax.ShapeDtypeStruct((N, N), _F8),
                   jax.ShapeDtypeStruct((N, F + _PAD), _F8),
                   jax.ShapeDtypeStruct((1, F), jnp.float32)),
        scratch_shapes=[pltpu.VMEM((N, F), jnp.bfloat16)],
    )(x, W1, b1.reshape(1, F), adj, Wc1)

    out = pl.pallas_call(
        _final_kernel,
        grid=(N // _BM2,),
        in_specs=[pl.BlockSpec((_BM2, N), lambda i: (i, 0)),
                  full((N, F + _PAD)),
                  full((1, F)),
                  full((F, F)), full((F, C)), full((1, C))],
        out_specs=pl.BlockSpec((_BM2, C), lambda i: (i, 0)),
        out_shape=jax.ShapeDtypeStruct((N, C), jnp.float32),
    )(adj8, s2_8, d, Wc2, W2, b2.reshape(1, C))

    return out
